# baseline (speedup 1.0000x reference)
import jax
import jax.numpy as jnp
from jax.experimental import pallas as pl

_H = 3
_F = 64


def _leaky(x):
    return jnp.where(x >= 0, x, 0.01 * x)


def _lin_kernel(x_ref, w_ref, b_ref, o_ref):
    o_ref[...] = (
        jnp.dot(x_ref[...], w_ref[...], preferred_element_type=jnp.float32)
        + b_ref[...]
    )


def _pallas_linear(x, w, b, block):
    n, d = x.shape
    f = w.shape[1]
    return pl.pallas_call(
        _lin_kernel,
        grid=(n // block,),
        in_specs=[
            pl.BlockSpec((block, d), lambda i: (i, 0)),
            pl.BlockSpec((d, f), lambda i: (0, 0)),
            pl.BlockSpec((1, f), lambda i: (0, 0)),
        ],
        out_specs=pl.BlockSpec((block, f), lambda i: (i, 0)),
        out_shape=jax.ShapeDtypeStruct((n, f), jnp.float32),
    )(x, w, b)


def _heads_kernel(x_ref, gw1_ref, gb1_ref, gw2_ref, gb2_ref,
                  mw1_ref, mb1_ref, mw2_ref, mb2_ref, gate_ref, msg_ref):
    x = x_ref[...]
    gates = []
    msgs = []
    for h in range(_H):
        hg = _leaky(
            jnp.dot(x, gw1_ref[h], preferred_element_type=jnp.float32)
            + gb1_ref[h]
        )
        g = (
            jnp.dot(hg, gw2_ref[h], preferred_element_type=jnp.float32)
            + gb2_ref[h]
        )
        hm = _leaky(
            jnp.dot(x, mw1_ref[h], preferred_element_type=jnp.float32)
            + mb1_ref[h]
        )
        m = (
            jnp.dot(hm, mw2_ref[h], preferred_element_type=jnp.float32)
            + mb2_ref[h]
        )
        gates.append(g)
        msgs.append(m)
    gate_ref[...] = jnp.concatenate(gates, axis=1)
    msg_ref[...] = jnp.concatenate(msgs, axis=1)


def _pallas_heads(x, wts, block):
    n, d = x.shape
    gw1, gb1, gw2, gb2, mw1, mb1, mw2, mb2 = wts
    hid = gw1.shape[2]
    f = mw2.shape[2]
    wspecs = [
        pl.BlockSpec((_H, d, hid), lambda i: (0, 0, 0)),
        pl.BlockSpec((_H, 1, hid), lambda i: (0, 0, 0)),
        pl.BlockSpec((_H, hid, 1), lambda i: (0, 0, 0)),
        pl.BlockSpec((_H, 1, 1), lambda i: (0, 0, 0)),
        pl.BlockSpec((_H, d, hid), lambda i: (0, 0, 0)),
        pl.BlockSpec((_H, 1, hid), lambda i: (0, 0, 0)),
        pl.BlockSpec((_H, hid, f), lambda i: (0, 0, 0)),
        pl.BlockSpec((_H, 1, f), lambda i: (0, 0, 0)),
    ]
    return pl.pallas_call(
        _heads_kernel,
        grid=(n // block,),
        in_specs=[pl.BlockSpec((block, d), lambda i: (i, 0))] + wspecs,
        out_specs=[
            pl.BlockSpec((block, _H), lambda i: (i, 0)),
            pl.BlockSpec((block, _H * f), lambda i: (i, 0)),
        ],
        out_shape=[
            jax.ShapeDtypeStruct((n, _H), jnp.float32),
            jax.ShapeDtypeStruct((n, _H * f), jnp.float32),
        ],
    )(x, gw1, gb1, gw2, gb2, mw1, mb1, mw2, mb2)


def _stack_pool(layer):
    gw1 = jnp.stack([p["gate"][0]["W"] for p in layer])
    gb1 = jnp.stack([p["gate"][0]["b"][None, :] for p in layer])
    gw2 = jnp.stack([p["gate"][1]["W"] for p in layer])
    gb2 = jnp.stack([p["gate"][1]["b"][None, :] for p in layer])
    mw1 = jnp.stack([p["msg"][0]["W"] for p in layer])
    mb1 = jnp.stack([p["msg"][0]["b"][None, :] for p in layer])
    mw2 = jnp.stack([p["msg"][1]["W"] for p in layer])
    mb2 = jnp.stack([p["msg"][1]["b"][None, :] for p in layer])
    pows = jnp.stack([p["pow"] for p in layer])
    return (gw1, gb1, gw2, gb2, mw1, mb1, mw2, mb2), pows


def _attn_pool(gate_all, msg_all, index, weights, pows, num_segments):
    out = None
    for h in range(_H):
        g = gate_all[:, h:h + 1]
        seg_max = jax.ops.segment_max(g, index, num_segments=num_segments)
        g = g - seg_max[index]
        g = (weights ** pows[h]) * jnp.exp(g)
        denom = jax.ops.segment_sum(g, index, num_segments=num_segments)
        g = g / (denom[index] + 1e-10)
        m = msg_all[:, h * _F:(h + 1) * _F]
        seg = jax.ops.segment_sum(g * m, index, num_segments=num_segments)
        out = seg if out is None else out + seg
    return out / _H


def kernel(elem_weights, elem_fea, self_idx, nbr_idx, cry_elem_idx, params):
    n_nodes = elem_fea.shape[0]
    n_cry = 2000

    emb = params["embedding"]
    w = jnp.pad(emb["W"], ((0, 0), (0, 1)))
    b = jnp.pad(emb["b"], (0, 1))[None, :]
    fea = _pallas_linear(elem_fea, w, b, 2000)[:, : _F - 1]
    fea = jnp.concatenate([fea, elem_weights], axis=1)

    nbr_weights = elem_weights[nbr_idx]
    for layer in params["graphs"]:
        pair = jnp.concatenate([fea[self_idx], fea[nbr_idx]], axis=1)
        wts, pows = _stack_pool(layer)
        gate_all, msg_all = _pallas_heads(pair, wts, 6400)
        agg = _attn_pool(gate_all, msg_all, self_idx, nbr_weights, pows,
                         n_nodes)
        fea = agg + fea

    wts, pows = _stack_pool(params["cry_pool"])
    gate_all, msg_all = _pallas_heads(fea, wts, 2000)
    return _attn_pool(gate_all, msg_all, cry_elem_idx, elem_weights, pows,
                      n_cry)
